# baseline (speedup 1.0000x reference)
import functools
import math

import jax
import jax.numpy as jnp
from jax import lax
from jax.experimental import pallas as pl
from jax.experimental.pallas import tpu as pltpu
from jax.experimental.pallas import tpu_sc as plsc

_D = 64
_SCALE = math.sqrt(_D)
_CH = 128


@functools.lru_cache(maxsize=None)
def _make_kernel(B: int):
    info = plsc.get_sparse_core_info()
    NC, NS, L = info.num_cores, info.num_subcores, info.num_lanes
    NW = NC * NS
    assert B % (NW * _CH) == 0
    b_per_w = B // NW
    n_ch = b_per_w // _CH
    mesh = plsc.VectorSubcoreMesh(core_axis_name="c", subcore_axis_name="s")

    @functools.partial(
        pl.kernel,
        mesh=mesh,
        compiler_params=pltpu.CompilerParams(use_tc_tiling_on_sc=False),
        out_type=jax.ShapeDtypeStruct((B, _D), jnp.float32),
        scratch_types=[
            pltpu.VMEM((_CH,), jnp.int32),
            pltpu.VMEM((_CH, _D), jnp.float32),
            pltpu.SemaphoreType.DMA,
        ],
    )
    def emb_kernel(idx_hbm, table_hbm, out_hbm, idx_v, rows_v, sem):
        wid = lax.axis_index("s") * NC + lax.axis_index("c")
        base = wid * b_per_w

        def chunk(g, carry):
            off = base + g * _CH
            pltpu.sync_copy(idx_hbm.at[pl.ds(off, _CH)], idx_v)
            pltpu.async_copy(table_hbm.at[idx_v], rows_v, sem).wait()

            def row(r, c2):
                for c in range(_D // L):
                    sl = pl.ds(c * L, L)
                    rows_v[r, sl] = rows_v[r, sl] * _SCALE
                return c2

            lax.fori_loop(0, _CH, row, 0)
            pltpu.sync_copy(rows_v, out_hbm.at[pl.ds(off, _CH)])
            return carry

        lax.fori_loop(0, n_ch, chunk, 0)

    return emb_kernel


def kernel(x, emb_weight):
    s0, s1 = x.shape
    b = s0 * s1
    flat_idx = jnp.reshape(x, (b,)).astype(jnp.int32)
    out = _make_kernel(b)(flat_idx, emb_weight)
    return jnp.reshape(out, (s0, s1, _D))


# speedup vs baseline: 1.2576x; 1.2576x over previous
import functools
import math

import jax
import jax.numpy as jnp
from jax import lax
from jax.experimental import pallas as pl
from jax.experimental.pallas import tpu as pltpu
from jax.experimental.pallas import tpu_sc as plsc

_D = 64
_SCALE = math.sqrt(_D)
_GCH = 128
_KB = 4
_SCH = _GCH * _KB
_NBUF = 2


@functools.lru_cache(maxsize=None)
def _make_kernel(B: int):
    info = plsc.get_sparse_core_info()
    NC, NS, L = info.num_cores, info.num_subcores, info.num_lanes
    NW = NC * NS
    assert B % (NW * _SCH) == 0
    b_per_w = B // NW
    n_super = b_per_w // _SCH
    assert n_super % _NBUF == 0
    mesh = plsc.VectorSubcoreMesh(core_axis_name="c", subcore_axis_name="s")

    @functools.partial(
        pl.kernel,
        mesh=mesh,
        compiler_params=pltpu.CompilerParams(use_tc_tiling_on_sc=False),
        out_type=jax.ShapeDtypeStruct((B, _D), jnp.float32),
        scratch_types=[
            pltpu.VMEM((_NBUF, _KB, _GCH), jnp.int32),
            pltpu.VMEM((_NBUF, _SCH, _D), jnp.float32),
            pltpu.SemaphoreType.DMA((_NBUF,)),
            pltpu.SemaphoreType.DMA((_NBUF,)),
        ],
    )
    def emb_kernel(idx_hbm, table_hbm, out_hbm, idx_v, rows_v, gsem, ssem):
        wid = lax.axis_index("s") * NC + lax.axis_index("c")
        base = wid * b_per_w
        ibase = wid * (b_per_w // _GCH)

        def stage(s, b):
            pltpu.sync_copy(idx_hbm.at[pl.ds(ibase + s * _KB, _KB)],
                            idx_v.at[b])
            for j in range(_KB):
                pltpu.async_copy(table_hbm.at[idx_v.at[b, j]],
                                 rows_v.at[b, pl.ds(j * _GCH, _GCH)],
                                 gsem.at[b])

        def drain(sem, b):
            pltpu.make_async_copy(out_hbm.at[pl.ds(0, _SCH)],
                                  rows_v.at[b], sem.at[b]).wait()

        stage(0, 0)

        def body(s):
            for b in range(_NBUF):
                cur = s + b
                nb = 1 - b

                @pl.when(cur + 1 < n_super)
                def _():
                    @pl.when(cur >= 1)
                    def _():
                        drain(ssem, nb)
                    stage(cur + 1, nb)

                drain(gsem, b)

                def scale(r):
                    for c in range(_D // L):
                        sl = pl.ds(c * L, L)
                        rows_v[b, r, sl] = rows_v[b, r, sl] * _SCALE

                pl.loop(0, _SCH, unroll=8)(scale)
                pltpu.async_copy(rows_v.at[b],
                                 out_hbm.at[pl.ds(base + cur * _SCH, _SCH)],
                                 ssem.at[b])

        pl.loop(0, n_super, step=_NBUF)(body)
        for b in range(_NBUF):
            drain(ssem, b)

    return emb_kernel


def kernel(x, emb_weight):
    s0, s1 = x.shape
    b = s0 * s1
    flat_idx = jnp.reshape(x, (b // _GCH, _GCH)).astype(jnp.int32)
    out = _make_kernel(b)(flat_idx, emb_weight)
    return jnp.reshape(out, (s0, s1, _D))
